# baseline (speedup 1.0000x reference)
import functools

import jax
import jax.numpy as jnp
from jax import lax
from jax.experimental import pallas as pl
from jax.experimental.pallas import tpu as pltpu
from jax.experimental.pallas import tpu_sc as plsc

N = 10000
E = 320000
H = 128
G = 64
C = 10
EPS_BN = 1e-5

NC = 2
NS = 16
NW = NC * NS
K = 128
CH = 80
EPW = CH * K
NROW = N + 8
RPT = 624
TAIL = N - NS * RPT

_mesh = plsc.VectorSubcoreMesh(core_axis_name="c", subcore_axis_name="s")


@functools.partial(
    pl.kernel,
    out_type=jax.ShapeDtypeStruct((NC, N, H), jnp.float32),
    mesh=_mesh,
    scratch_types=[
        pltpu.VMEM((CH, K), jnp.int32),
        pltpu.VMEM((K,), jnp.int32),
        pltpu.VMEM((K,), jnp.int32),
        pltpu.VMEM((K, H), jnp.float32),
        pltpu.VMEM((K, H), jnp.float32),
        pltpu.VMEM_SHARED((NROW, H), jnp.float32),
        pltpu.SemaphoreType.DMA,
        pltpu.SemaphoreType.DMA,
        pltpu.SemaphoreType.DMA,
        pltpu.SemaphoreType.DMA,
    ],
)
def _sc_agg(h_hbm, src_hbm, dst_hbm, zeros_hbm, out_hbm,
            src_v, d0, d1, buf0, buf1, agg_sh, g0, g1, di0, di1):
    c = lax.axis_index("c")
    s = lax.axis_index("s")
    r0 = s * RPT
    pltpu.sync_copy(zeros_hbm.at[pl.ds(r0, RPT)], agg_sh.at[pl.ds(r0, RPT)])

    @pl.when(s == 0)
    def _zero_tail():
        pltpu.sync_copy(zeros_hbm.at[pl.ds(NS * RPT, TAIL)],
                        agg_sh.at[pl.ds(NS * RPT, TAIL)])
    pltpu.sync_copy(src_hbm.at[c, s], src_v)
    pltpu.sync_copy(dst_hbm.at[c, s, 0], d0)
    plsc.subcore_barrier()

    pltpu.async_copy(h_hbm.at[src_v.at[0]], buf0, g0)

    def body(j, carry):
        i0 = 2 * j
        i1 = i0 + 1

        @pl.when(j > 0)
        def _wait_d0():
            pltpu.make_async_copy(dst_hbm.at[c, s, i0], d0, di0).wait()

        pltpu.async_copy(dst_hbm.at[c, s, i1], d1, di1)
        pltpu.make_async_copy(h_hbm.at[src_v.at[i0]], buf0, g0).wait()
        pltpu.async_copy(h_hbm.at[src_v.at[i1]], buf1, g1)
        pltpu.sync_copy(buf0, agg_sh.at[d0], add=True)

        @pl.when(i1 + 1 < CH)
        def _prefetch_d0():
            pltpu.async_copy(dst_hbm.at[c, s, i1 + 1], d0, di0)

        pltpu.make_async_copy(h_hbm.at[src_v.at[i1]], buf1, g1).wait()

        @pl.when(i1 + 1 < CH)
        def _next_gather():
            pltpu.async_copy(h_hbm.at[src_v.at[i1 + 1]], buf0, g0)

        pltpu.make_async_copy(dst_hbm.at[c, s, i1], d1, di1).wait()
        pltpu.sync_copy(buf1, agg_sh.at[d1], add=True)
        return carry

    lax.fori_loop(0, CH // 2, body, 0)
    plsc.subcore_barrier()
    pltpu.sync_copy(agg_sh.at[pl.ds(r0, RPT)], out_hbm.at[c, pl.ds(r0, RPT)])

    @pl.when(s == 0)
    def _write_tail():
        pltpu.sync_copy(agg_sh.at[pl.ds(NS * RPT, TAIL)],
                        out_hbm.at[c, pl.ds(NS * RPT, TAIL)])


BN = 2000
NB = N // BN


def _mlp_body(h_ref, p_ref, w0_ref, b0_ref, w1_ref, b1_ref, out_ref):
    u = h_ref[...] + p_ref[0] + p_ref[1]
    y = jnp.dot(u, w0_ref[...], preferred_element_type=jnp.float32) + b0_ref[...]
    y = jnp.where(y >= 0.0, y, 0.1 * y)
    out_ref[...] = jnp.dot(y, w1_ref[...], preferred_element_type=jnp.float32) + b1_ref[...]


def _mlp(h, parts, w0t, b0, w1t, b1):
    return pl.pallas_call(
        _mlp_body,
        out_shape=jax.ShapeDtypeStruct((N, H), jnp.float32),
        grid=(NB,),
        in_specs=[
            pl.BlockSpec((BN, H), lambda i: (i, 0)),
            pl.BlockSpec((NC, BN, H), lambda i: (0, i, 0)),
            pl.BlockSpec((H, H), lambda i: (0, 0)),
            pl.BlockSpec((1, H), lambda i: (0, 0)),
            pl.BlockSpec((H, H), lambda i: (0, 0)),
            pl.BlockSpec((1, H), lambda i: (0, 0)),
        ],
        out_specs=pl.BlockSpec((BN, H), lambda i: (i, 0)),
    )(h, parts, w0t, b0, w1t, b1)


def _pool_body(b_ref, h_ref, w0_ref, b0_ref, w1_ref, b1_ref, out_ref, acc_ref):
    i = pl.program_id(0)

    @pl.when(i == 0)
    def _init():
        acc_ref[...] = jnp.zeros_like(acc_ref)

    seg = b_ref[0]
    onehot = (lax.broadcasted_iota(jnp.int32, (G, BN), 0) == seg).astype(jnp.float32)
    acc_ref[...] += jnp.dot(onehot, h_ref[...], preferred_element_type=jnp.float32)

    @pl.when(i == NB - 1)
    def _fin():
        y = jnp.dot(acc_ref[...], w0_ref[...], preferred_element_type=jnp.float32) + b0_ref[...]
        y = jnp.where(y >= 0.0, y, 0.1 * y)
        out_ref[...] = jnp.dot(y, w1_ref[...], preferred_element_type=jnp.float32) + b1_ref[...]


def _pool(batch3, h, w0t, b0, w1t, b1):
    return pl.pallas_call(
        _pool_body,
        out_shape=jax.ShapeDtypeStruct((G, H), jnp.float32),
        grid=(NB,),
        in_specs=[
            pl.BlockSpec((1, 1, BN), lambda i: (i, 0, 0)),
            pl.BlockSpec((BN, H), lambda i: (i, 0)),
            pl.BlockSpec((H, H), lambda i: (0, 0)),
            pl.BlockSpec((1, H), lambda i: (0, 0)),
            pl.BlockSpec((H, H), lambda i: (0, 0)),
            pl.BlockSpec((1, H), lambda i: (0, 0)),
        ],
        out_specs=pl.BlockSpec((G, H), lambda i: (0, 0)),
        scratch_shapes=[pltpu.VMEM((G, H), jnp.float32)],
    )(batch3, h, w0t, b0, w1t, b1)


def _fold_bn(W0, b0, gam, bet, W1, b1):
    scale = gam / jnp.sqrt(1.0 + EPS_BN)
    w0t = (W0 * scale[:, None]).T
    b0e = (b0 * scale + bet)[None, :]
    return w0t, b0e, W1.T, b1[None, :]


def kernel(x, edge_index, batch,
           Wg00, bg00, gam0, bet0, Wg01, bg01,
           Wg10, bg10, gam1, bet1, Wg11, bg11,
           Wg20, bg20, gam2, bet2, Wg21, bg21,
           Wp0, bp0, gamp, betp, Wp1, bp1):
    pad = NW * EPW - E
    src = jnp.concatenate([edge_index[0], jnp.zeros((pad,), jnp.int32)])
    src = src.reshape(NC, NS, CH, K)
    dst = jnp.concatenate([edge_index[1], jnp.full((pad,), N, jnp.int32)])
    dst = dst.reshape(NC, NS, CH, K)
    zeros = jnp.zeros((N, H), jnp.float32)

    layers = [
        _fold_bn(Wg00, bg00, gam0, bet0, Wg01, bg01),
        _fold_bn(Wg10, bg10, gam1, bet1, Wg11, bg11),
        _fold_bn(Wg20, bg20, gam2, bet2, Wg21, bg21),
    ]

    h = x
    for w0t, b0e, w1t, b1e in layers:
        parts = _sc_agg(h, src, dst, zeros)
        h = _mlp(h, parts, w0t, b0e, w1t, b1e)

    batch3 = batch.reshape(NB, 1, BN)
    scalep = gamp / jnp.sqrt(1.0 + EPS_BN)
    wp0t = (Wp0 * scalep[:, None]).T
    bp0e = (bp0 * scalep + betp)[None, :]
    wp1t = jnp.zeros((H, H), jnp.float32).at[:, :C].set(Wp1.T)
    bp1e = jnp.zeros((1, H), jnp.float32).at[0, :C].set(bp1)

    y = _pool(batch3, h, wp0t, bp0e, wp1t, bp1e)
    return y[:, :C]


# speedup vs baseline: 2.7288x; 2.7288x over previous
import functools

import jax
import jax.numpy as jnp
from jax import lax
from jax.experimental import pallas as pl
from jax.experimental.pallas import tpu as pltpu
from jax.experimental.pallas import tpu_sc as plsc

N = 10000
E = 320000
H = 128
G = 64
C = 10
EPS_BN = 1e-5

NC = 2
NS = 16
NW = NC * NS
K = 80
CH = (E // NW) // K
NROW = N
RPT = 624
TAIL = N - NS * RPT

_mesh = plsc.VectorSubcoreMesh(core_axis_name="c", subcore_axis_name="s")


@functools.partial(
    pl.kernel,
    out_type=jax.ShapeDtypeStruct((NC, N, H), jnp.float32),
    mesh=_mesh,
    scratch_types=[
        pltpu.VMEM((CH, K), jnp.int32),
        pltpu.VMEM((K,), jnp.int32),
        pltpu.VMEM((K,), jnp.int32),
        pltpu.VMEM((K, H), jnp.float32),
        pltpu.VMEM((K, H), jnp.float32),
        pltpu.VMEM_SHARED((NROW, H), jnp.float32),
        pltpu.SemaphoreType.DMA,
        pltpu.SemaphoreType.DMA,
        pltpu.SemaphoreType.DMA,
        pltpu.SemaphoreType.DMA,
    ],
)
def _sc_agg(h_hbm, src_hbm, dst_hbm, zeros_hbm, out_hbm,
            src_v, d0, d1, buf0, buf1, agg_sh, g0, g1, di0, di1):
    c = lax.axis_index("c")
    s = lax.axis_index("s")
    eb = (c * NS + s) * (CH * K)
    r0 = s * RPT
    pltpu.sync_copy(zeros_hbm.at[pl.ds(r0, RPT)], agg_sh.at[pl.ds(r0, RPT)])

    @pl.when(s == 0)
    def _zero_tail():
        pltpu.sync_copy(zeros_hbm.at[pl.ds(NS * RPT, TAIL)],
                        agg_sh.at[pl.ds(NS * RPT, TAIL)])
    pltpu.sync_copy(src_hbm.at[c, s], src_v)
    pltpu.sync_copy(dst_hbm.at[pl.ds(eb, K)], d0)
    plsc.subcore_barrier()

    pltpu.async_copy(h_hbm.at[src_v.at[0]], buf0, g0)

    def body(j, carry):
        i0 = 2 * j
        i1 = i0 + 1

        @pl.when(j > 0)
        def _wait_d0():
            pltpu.make_async_copy(dst_hbm.at[pl.ds(eb + i0 * K, K)], d0, di0).wait()

        pltpu.async_copy(dst_hbm.at[pl.ds(eb + i1 * K, K)], d1, di1)
        pltpu.make_async_copy(h_hbm.at[src_v.at[i0]], buf0, g0).wait()
        pltpu.async_copy(h_hbm.at[src_v.at[i1]], buf1, g1)
        pltpu.sync_copy(buf0, agg_sh.at[d0], add=True)
        pltpu.async_copy(dst_hbm.at[pl.ds(eb + (i1 + 1) * K, K)], d0, di0)
        pltpu.make_async_copy(h_hbm.at[src_v.at[i1]], buf1, g1).wait()
        pltpu.async_copy(h_hbm.at[src_v.at[i1 + 1]], buf0, g0)
        pltpu.make_async_copy(dst_hbm.at[pl.ds(eb + i1 * K, K)], d1, di1).wait()
        pltpu.sync_copy(buf1, agg_sh.at[d1], add=True)
        return carry

    lax.fori_loop(0, CH // 2, body, 0)
    pltpu.make_async_copy(dst_hbm.at[pl.ds(eb + (CH - 1) * K, K)], d0, di0).wait()
    pltpu.make_async_copy(h_hbm.at[src_v.at[CH - 1]], buf0, g0).wait()
    pltpu.sync_copy(buf0, agg_sh.at[d0], add=True)
    plsc.subcore_barrier()
    pltpu.sync_copy(agg_sh.at[pl.ds(r0, RPT)], out_hbm.at[c, pl.ds(r0, RPT)])

    @pl.when(s == 0)
    def _write_tail():
        pltpu.sync_copy(agg_sh.at[pl.ds(NS * RPT, TAIL)],
                        out_hbm.at[c, pl.ds(NS * RPT, TAIL)])


BN = 2000
NB = N // BN


def _mlp_body(h_ref, p_ref, w0_ref, b0_ref, w1_ref, b1_ref, out_ref):
    u = h_ref[...] + p_ref[0] + p_ref[1]
    y = jnp.dot(u, w0_ref[...], preferred_element_type=jnp.float32) + b0_ref[...]
    y = jnp.where(y >= 0.0, y, 0.1 * y)
    out_ref[...] = jnp.dot(y, w1_ref[...], preferred_element_type=jnp.float32) + b1_ref[...]


def _mlp(h, parts, w0t, b0, w1t, b1):
    return pl.pallas_call(
        _mlp_body,
        out_shape=jax.ShapeDtypeStruct((N, H), jnp.float32),
        grid=(NB,),
        in_specs=[
            pl.BlockSpec((BN, H), lambda i: (i, 0)),
            pl.BlockSpec((NC, BN, H), lambda i: (0, i, 0)),
            pl.BlockSpec((H, H), lambda i: (0, 0)),
            pl.BlockSpec((1, H), lambda i: (0, 0)),
            pl.BlockSpec((H, H), lambda i: (0, 0)),
            pl.BlockSpec((1, H), lambda i: (0, 0)),
        ],
        out_specs=pl.BlockSpec((BN, H), lambda i: (i, 0)),
    )(h, parts, w0t, b0, w1t, b1)


def _pool_body(b_ref, h_ref, w0_ref, b0_ref, w1_ref, b1_ref, out_ref, acc_ref):
    i = pl.program_id(0)

    @pl.when(i == 0)
    def _init():
        acc_ref[...] = jnp.zeros_like(acc_ref)

    seg = b_ref[0]
    onehot = (lax.broadcasted_iota(jnp.int32, (G, BN), 0) == seg).astype(jnp.float32)
    acc_ref[...] += jnp.dot(onehot, h_ref[...], preferred_element_type=jnp.float32)

    @pl.when(i == NB - 1)
    def _fin():
        y = jnp.dot(acc_ref[...], w0_ref[...], preferred_element_type=jnp.float32) + b0_ref[...]
        y = jnp.where(y >= 0.0, y, 0.1 * y)
        out_ref[...] = jnp.dot(y, w1_ref[...], preferred_element_type=jnp.float32) + b1_ref[...]


def _pool(batch3, h, w0t, b0, w1t, b1):
    return pl.pallas_call(
        _pool_body,
        out_shape=jax.ShapeDtypeStruct((G, H), jnp.float32),
        grid=(NB,),
        in_specs=[
            pl.BlockSpec((1, 1, BN), lambda i: (i, 0, 0)),
            pl.BlockSpec((BN, H), lambda i: (i, 0)),
            pl.BlockSpec((H, H), lambda i: (0, 0)),
            pl.BlockSpec((1, H), lambda i: (0, 0)),
            pl.BlockSpec((H, H), lambda i: (0, 0)),
            pl.BlockSpec((1, H), lambda i: (0, 0)),
        ],
        out_specs=pl.BlockSpec((G, H), lambda i: (0, 0)),
        scratch_shapes=[pltpu.VMEM((G, H), jnp.float32)],
    )(batch3, h, w0t, b0, w1t, b1)


def _fold_bn(W0, b0, gam, bet, W1, b1):
    scale = gam / jnp.sqrt(1.0 + EPS_BN)
    w0t = (W0 * scale[:, None]).T
    b0e = (b0 * scale + bet)[None, :]
    return w0t, b0e, W1.T, b1[None, :]


def kernel(x, edge_index, batch,
           Wg00, bg00, gam0, bet0, Wg01, bg01,
           Wg10, bg10, gam1, bet1, Wg11, bg11,
           Wg20, bg20, gam2, bet2, Wg21, bg21,
           Wp0, bp0, gamp, betp, Wp1, bp1):
    src = edge_index[0].reshape(NC, NS, CH, K)
    dst = edge_index[1]
    zeros = jnp.zeros((N, H), jnp.float32)

    layers = [
        _fold_bn(Wg00, bg00, gam0, bet0, Wg01, bg01),
        _fold_bn(Wg10, bg10, gam1, bet1, Wg11, bg11),
        _fold_bn(Wg20, bg20, gam2, bet2, Wg21, bg21),
    ]

    h = x
    for w0t, b0e, w1t, b1e in layers:
        parts = _sc_agg(h, src, dst, zeros)
        h = _mlp(h, parts, w0t, b0e, w1t, b1e)

    batch3 = batch.reshape(NB, 1, BN)
    scalep = gamp / jnp.sqrt(1.0 + EPS_BN)
    wp0t = (Wp0 * scalep[:, None]).T
    bp0e = (bp0 * scalep + betp)[None, :]
    wp1t = jnp.zeros((H, H), jnp.float32).at[:, :C].set(Wp1.T)
    bp1e = jnp.zeros((1, H), jnp.float32).at[0, :C].set(bp1)

    y = _pool(batch3, h, wp0t, bp0e, wp1t, bp1e)
    return y[:, :C]


# speedup vs baseline: 3.9640x; 1.4527x over previous
import functools

import jax
import jax.numpy as jnp
from jax import lax
from jax.experimental import pallas as pl
from jax.experimental.pallas import tpu as pltpu
from jax.experimental.pallas import tpu_sc as plsc

N = 10000
E = 320000
H = 128
G = 64
C = 10
EPS_BN = 1e-5

NC = 2
NS = 16
NW = NC * NS
K = 80
CH = (E // NW) // K
NROW = N
RPT = 624
TAIL = N - NS * RPT

_mesh = plsc.VectorSubcoreMesh(core_axis_name="c", subcore_axis_name="s")


@functools.partial(
    pl.kernel,
    out_type=jax.ShapeDtypeStruct((NC, N, H), jnp.float32),
    mesh=_mesh,
    scratch_types=[
        pltpu.VMEM((CH, K), jnp.int32),
        pltpu.VMEM((K,), jnp.int32),
        pltpu.VMEM((K,), jnp.int32),
        pltpu.VMEM((K,), jnp.int32),
        pltpu.VMEM((K, H), jnp.float32),
        pltpu.VMEM((K, H), jnp.float32),
        pltpu.VMEM((K, H), jnp.float32),
        pltpu.VMEM_SHARED((NROW, H), jnp.float32),
        pltpu.SemaphoreType.DMA,
        pltpu.SemaphoreType.DMA,
        pltpu.SemaphoreType.DMA,
        pltpu.SemaphoreType.DMA,
        pltpu.SemaphoreType.DMA,
        pltpu.SemaphoreType.DMA,
        pltpu.SemaphoreType.DMA,
        pltpu.SemaphoreType.DMA,
        pltpu.SemaphoreType.DMA,
    ],
)
def _sc_agg(h_hbm, src_hbm, dst_hbm, zeros_hbm, out_hbm,
            src_v, d0, d1, d2, buf0, buf1, buf2, agg_sh,
            g0, g1, g2, di0, di1, di2, ss0, ss1, ss2):
    c = lax.axis_index("c")
    s = lax.axis_index("s")
    eb = (c * NS + s) * (CH * K)
    r0 = s * RPT
    pltpu.sync_copy(zeros_hbm.at[pl.ds(r0, RPT)], agg_sh.at[pl.ds(r0, RPT)])

    @pl.when(s == 0)
    def _zero_tail():
        pltpu.sync_copy(zeros_hbm.at[pl.ds(NS * RPT, TAIL)],
                        agg_sh.at[pl.ds(NS * RPT, TAIL)])
    pltpu.sync_copy(src_hbm.at[c, s], src_v)
    plsc.subcore_barrier()

    D = (d0, d1, d2)
    B = (buf0, buf1, buf2)
    GS = (g0, g1, g2)
    DI = (di0, di1, di2)
    SS = (ss0, ss1, ss2)

    def wait_gather(i, p):
        pltpu.make_async_copy(h_hbm.at[src_v.at[i]], B[p], GS[p]).wait()

    def wait_didx(i, p):
        pltpu.make_async_copy(dst_hbm.at[pl.ds(eb + i * K, K)], D[p], DI[p]).wait()

    def wait_scat(p):
        pltpu.make_async_copy(B[p], agg_sh.at[D[p]], SS[p]).wait()

    def issue(i, p):
        pltpu.async_copy(dst_hbm.at[pl.ds(eb + i * K, K)], D[p], DI[p])
        pltpu.async_copy(h_hbm.at[src_v.at[i]], B[p], GS[p])

    issue(0, 0)

    def step(i, p, q, prefetch):
        @pl.when(i >= 2)
        def _drain():
            wait_scat(q)

        if prefetch:
            issue(i + 1, q)
        wait_didx(i, p)
        wait_gather(i, p)
        pltpu.async_copy(B[p], agg_sh.at[D[p]], SS[p], add=True)

    def body(j, carry):
        i = 3 * j
        step(i, 0, 1, True)
        step(i + 1, 1, 2, True)
        step(i + 2, 2, 0, True)
        return carry

    lax.fori_loop(0, (CH - 2) // 3, body, 0)
    step(CH - 2, 0, 1, True)
    step(CH - 1, 1, 2, False)
    wait_scat(0)
    wait_scat(1)
    plsc.subcore_barrier()
    pltpu.sync_copy(agg_sh.at[pl.ds(r0, RPT)], out_hbm.at[c, pl.ds(r0, RPT)])

    @pl.when(s == 0)
    def _write_tail():
        pltpu.sync_copy(agg_sh.at[pl.ds(NS * RPT, TAIL)],
                        out_hbm.at[c, pl.ds(NS * RPT, TAIL)])


BN = 2000
NB = N // BN


def _mlp_body(h_ref, p_ref, w0_ref, b0_ref, w1_ref, b1_ref, out_ref):
    u = h_ref[...] + p_ref[0] + p_ref[1]
    y = jnp.dot(u, w0_ref[...], preferred_element_type=jnp.float32) + b0_ref[...]
    y = jnp.where(y >= 0.0, y, 0.1 * y)
    out_ref[...] = jnp.dot(y, w1_ref[...], preferred_element_type=jnp.float32) + b1_ref[...]


def _mlp(h, parts, w0t, b0, w1t, b1):
    return pl.pallas_call(
        _mlp_body,
        out_shape=jax.ShapeDtypeStruct((N, H), jnp.float32),
        grid=(NB,),
        in_specs=[
            pl.BlockSpec((BN, H), lambda i: (i, 0)),
            pl.BlockSpec((NC, BN, H), lambda i: (0, i, 0)),
            pl.BlockSpec((H, H), lambda i: (0, 0)),
            pl.BlockSpec((1, H), lambda i: (0, 0)),
            pl.BlockSpec((H, H), lambda i: (0, 0)),
            pl.BlockSpec((1, H), lambda i: (0, 0)),
        ],
        out_specs=pl.BlockSpec((BN, H), lambda i: (i, 0)),
    )(h, parts, w0t, b0, w1t, b1)


def _pool_body(b_ref, h_ref, w0_ref, b0_ref, w1_ref, b1_ref, out_ref, acc_ref):
    i = pl.program_id(0)

    @pl.when(i == 0)
    def _init():
        acc_ref[...] = jnp.zeros_like(acc_ref)

    seg = b_ref[0]
    onehot = (lax.broadcasted_iota(jnp.int32, (G, BN), 0) == seg).astype(jnp.float32)
    acc_ref[...] += jnp.dot(onehot, h_ref[...], preferred_element_type=jnp.float32)

    @pl.when(i == NB - 1)
    def _fin():
        y = jnp.dot(acc_ref[...], w0_ref[...], preferred_element_type=jnp.float32) + b0_ref[...]
        y = jnp.where(y >= 0.0, y, 0.1 * y)
        out_ref[...] = jnp.dot(y, w1_ref[...], preferred_element_type=jnp.float32) + b1_ref[...]


def _pool(batch3, h, w0t, b0, w1t, b1):
    return pl.pallas_call(
        _pool_body,
        out_shape=jax.ShapeDtypeStruct((G, H), jnp.float32),
        grid=(NB,),
        in_specs=[
            pl.BlockSpec((1, 1, BN), lambda i: (i, 0, 0)),
            pl.BlockSpec((BN, H), lambda i: (i, 0)),
            pl.BlockSpec((H, H), lambda i: (0, 0)),
            pl.BlockSpec((1, H), lambda i: (0, 0)),
            pl.BlockSpec((H, H), lambda i: (0, 0)),
            pl.BlockSpec((1, H), lambda i: (0, 0)),
        ],
        out_specs=pl.BlockSpec((G, H), lambda i: (0, 0)),
        scratch_shapes=[pltpu.VMEM((G, H), jnp.float32)],
    )(batch3, h, w0t, b0, w1t, b1)


def _fold_bn(W0, b0, gam, bet, W1, b1):
    scale = gam / jnp.sqrt(1.0 + EPS_BN)
    w0t = (W0 * scale[:, None]).T
    b0e = (b0 * scale + bet)[None, :]
    return w0t, b0e, W1.T, b1[None, :]


def kernel(x, edge_index, batch,
           Wg00, bg00, gam0, bet0, Wg01, bg01,
           Wg10, bg10, gam1, bet1, Wg11, bg11,
           Wg20, bg20, gam2, bet2, Wg21, bg21,
           Wp0, bp0, gamp, betp, Wp1, bp1):
    src = edge_index[0].reshape(NC, NS, CH, K)
    dst = edge_index[1]
    zeros = jnp.zeros((N, H), jnp.float32)

    layers = [
        _fold_bn(Wg00, bg00, gam0, bet0, Wg01, bg01),
        _fold_bn(Wg10, bg10, gam1, bet1, Wg11, bg11),
        _fold_bn(Wg20, bg20, gam2, bet2, Wg21, bg21),
    ]

    h = x
    for w0t, b0e, w1t, b1e in layers:
        parts = _sc_agg(h, src, dst, zeros)
        h = _mlp(h, parts, w0t, b0e, w1t, b1e)

    batch3 = batch.reshape(NB, 1, BN)
    scalep = gamp / jnp.sqrt(1.0 + EPS_BN)
    wp0t = (Wp0 * scalep[:, None]).T
    bp0e = (bp0 * scalep + betp)[None, :]
    wp1t = jnp.zeros((H, H), jnp.float32).at[:, :C].set(Wp1.T)
    bp1e = jnp.zeros((1, H), jnp.float32).at[0, :C].set(bp1)

    y = _pool(batch3, h, wp0t, bp0e, wp1t, bp1e)
    return y[:, :C]


# speedup vs baseline: 4.0714x; 1.0271x over previous
import functools

import jax
import jax.numpy as jnp
from jax import lax
from jax.experimental import pallas as pl
from jax.experimental.pallas import tpu as pltpu
from jax.experimental.pallas import tpu_sc as plsc

N = 10000
E = 320000
H = 128
G = 64
C = 10
EPS_BN = 1e-5

NC = 2
NS = 16
NW = NC * NS
K = 80
CH = (E // NW) // K
NROW = N
RPT = 624
TAIL = N - NS * RPT

_mesh = plsc.VectorSubcoreMesh(core_axis_name="c", subcore_axis_name="s")


@functools.partial(
    pl.kernel,
    out_type=jax.ShapeDtypeStruct((NC, N, H), jnp.float32),
    mesh=_mesh,
    scratch_types=[
        pltpu.VMEM((CH, K), jnp.int32),
        pltpu.VMEM((K,), jnp.int32),
        pltpu.VMEM((K,), jnp.int32),
        pltpu.VMEM((K,), jnp.int32),
        pltpu.VMEM((K, H), jnp.float32),
        pltpu.VMEM((K, H), jnp.float32),
        pltpu.VMEM((K, H), jnp.float32),
        pltpu.VMEM_SHARED((NROW, H), jnp.float32),
        pltpu.SemaphoreType.DMA,
        pltpu.SemaphoreType.DMA,
        pltpu.SemaphoreType.DMA,
        pltpu.SemaphoreType.DMA,
        pltpu.SemaphoreType.DMA,
        pltpu.SemaphoreType.DMA,
        pltpu.SemaphoreType.DMA,
        pltpu.SemaphoreType.DMA,
        pltpu.SemaphoreType.DMA,
    ],
)
def _sc_agg(h_hbm, src_hbm, dst_hbm, zeros_hbm, out_hbm,
            src_v, d0, d1, d2, buf0, buf1, buf2, agg_sh,
            g0, g1, g2, di0, di1, di2, ss0, ss1, ss2):
    c = lax.axis_index("c")
    s = lax.axis_index("s")
    eb = (c * NS + s) * (CH * K)
    r0 = s * RPT
    @pl.when(c == 0)
    def _init_h():
        pltpu.sync_copy(h_hbm.at[pl.ds(r0, RPT)], agg_sh.at[pl.ds(r0, RPT)])

    @pl.when(c == 1)
    def _init_z():
        pltpu.sync_copy(zeros_hbm.at[pl.ds(r0, RPT)], agg_sh.at[pl.ds(r0, RPT)])

    @pl.when((c == 0) & (s == 0))
    def _tail_h():
        pltpu.sync_copy(h_hbm.at[pl.ds(NS * RPT, TAIL)],
                        agg_sh.at[pl.ds(NS * RPT, TAIL)])

    @pl.when((c == 1) & (s == 0))
    def _tail_z():
        pltpu.sync_copy(zeros_hbm.at[pl.ds(NS * RPT, TAIL)],
                        agg_sh.at[pl.ds(NS * RPT, TAIL)])
    pltpu.sync_copy(src_hbm.at[c, s], src_v)
    plsc.subcore_barrier()

    D = (d0, d1, d2)
    B = (buf0, buf1, buf2)
    GS = (g0, g1, g2)
    DI = (di0, di1, di2)
    SS = (ss0, ss1, ss2)

    def wait_gather(i, p):
        pltpu.make_async_copy(h_hbm.at[src_v.at[i]], B[p], GS[p]).wait()

    def wait_didx(i, p):
        pltpu.make_async_copy(dst_hbm.at[pl.ds(eb + i * K, K)], D[p], DI[p]).wait()

    def wait_scat(p):
        pltpu.make_async_copy(B[p], agg_sh.at[D[p]], SS[p]).wait()

    def issue(i, p):
        pltpu.async_copy(dst_hbm.at[pl.ds(eb + i * K, K)], D[p], DI[p])
        pltpu.async_copy(h_hbm.at[src_v.at[i]], B[p], GS[p])

    issue(0, 0)

    def step(i, p, q, prefetch):
        @pl.when(i >= 2)
        def _drain():
            wait_scat(q)

        if prefetch:
            issue(i + 1, q)
        wait_didx(i, p)
        wait_gather(i, p)
        pltpu.async_copy(B[p], agg_sh.at[D[p]], SS[p], add=True)

    def body(j, carry):
        i = 3 * j
        step(i, 0, 1, True)
        step(i + 1, 1, 2, True)
        step(i + 2, 2, 0, True)
        return carry

    lax.fori_loop(0, (CH - 2) // 3, body, 0)
    step(CH - 2, 0, 1, True)
    step(CH - 1, 1, 2, False)
    wait_scat(0)
    wait_scat(1)
    plsc.subcore_barrier()
    pltpu.sync_copy(agg_sh.at[pl.ds(r0, RPT)], out_hbm.at[c, pl.ds(r0, RPT)])

    @pl.when(s == 0)
    def _write_tail():
        pltpu.sync_copy(agg_sh.at[pl.ds(NS * RPT, TAIL)],
                        out_hbm.at[c, pl.ds(NS * RPT, TAIL)])


BN = 2000
NB = N // BN


def _mlp_body(p_ref, w0_ref, b0_ref, w1_ref, b1_ref, out_ref):
    u = p_ref[0] + p_ref[1]
    y = jnp.dot(u, w0_ref[...], preferred_element_type=jnp.float32) + b0_ref[...]
    y = jnp.where(y >= 0.0, y, 0.1 * y)
    out_ref[...] = jnp.dot(y, w1_ref[...], preferred_element_type=jnp.float32) + b1_ref[...]


def _mlp(parts, w0t, b0, w1t, b1):
    return pl.pallas_call(
        _mlp_body,
        out_shape=jax.ShapeDtypeStruct((N, H), jnp.float32),
        grid=(NB,),
        in_specs=[
            pl.BlockSpec((NC, BN, H), lambda i: (0, i, 0)),
            pl.BlockSpec((H, H), lambda i: (0, 0)),
            pl.BlockSpec((1, H), lambda i: (0, 0)),
            pl.BlockSpec((H, H), lambda i: (0, 0)),
            pl.BlockSpec((1, H), lambda i: (0, 0)),
        ],
        out_specs=pl.BlockSpec((BN, H), lambda i: (i, 0)),
    )(parts, w0t, b0, w1t, b1)


def _mlp_pool_body(b_ref, p_ref, w0_ref, b0_ref, w1_ref, b1_ref,
                   wp0_ref, bp0_ref, wp1_ref, bp1_ref, out_ref, acc_ref):
    i = pl.program_id(0)

    @pl.when(i == 0)
    def _init():
        acc_ref[...] = jnp.zeros_like(acc_ref)

    u = p_ref[0] + p_ref[1]
    y = jnp.dot(u, w0_ref[...], preferred_element_type=jnp.float32) + b0_ref[...]
    y = jnp.where(y >= 0.0, y, 0.1 * y)
    h3 = jnp.dot(y, w1_ref[...], preferred_element_type=jnp.float32) + b1_ref[...]

    seg = b_ref[0]
    onehot = (lax.broadcasted_iota(jnp.int32, (G, BN), 0) == seg).astype(jnp.float32)
    acc_ref[...] += jnp.dot(onehot, h3, preferred_element_type=jnp.float32)

    @pl.when(i == NB - 1)
    def _fin():
        z = jnp.dot(acc_ref[...], wp0_ref[...], preferred_element_type=jnp.float32) + bp0_ref[...]
        z = jnp.where(z >= 0.0, z, 0.1 * z)
        out_ref[...] = jnp.dot(z, wp1_ref[...], preferred_element_type=jnp.float32) + bp1_ref[...]


def _mlp_pool(batch3, parts, w0t, b0, w1t, b1, wp0t, bp0, wp1t, bp1):
    return pl.pallas_call(
        _mlp_pool_body,
        out_shape=jax.ShapeDtypeStruct((G, H), jnp.float32),
        grid=(NB,),
        in_specs=[
            pl.BlockSpec((1, 1, BN), lambda i: (i, 0, 0)),
            pl.BlockSpec((NC, BN, H), lambda i: (0, i, 0)),
            pl.BlockSpec((H, H), lambda i: (0, 0)),
            pl.BlockSpec((1, H), lambda i: (0, 0)),
            pl.BlockSpec((H, H), lambda i: (0, 0)),
            pl.BlockSpec((1, H), lambda i: (0, 0)),
            pl.BlockSpec((H, H), lambda i: (0, 0)),
            pl.BlockSpec((1, H), lambda i: (0, 0)),
            pl.BlockSpec((H, H), lambda i: (0, 0)),
            pl.BlockSpec((1, H), lambda i: (0, 0)),
        ],
        out_specs=pl.BlockSpec((G, H), lambda i: (0, 0)),
        scratch_shapes=[pltpu.VMEM((G, H), jnp.float32)],
    )(batch3, parts, w0t, b0, w1t, b1, wp0t, bp0, wp1t, bp1)


def _fold_bn(W0, b0, gam, bet, W1, b1):
    scale = gam / jnp.sqrt(1.0 + EPS_BN)
    w0t = (W0 * scale[:, None]).T
    b0e = (b0 * scale + bet)[None, :]
    return w0t, b0e, W1.T, b1[None, :]


def kernel(x, edge_index, batch,
           Wg00, bg00, gam0, bet0, Wg01, bg01,
           Wg10, bg10, gam1, bet1, Wg11, bg11,
           Wg20, bg20, gam2, bet2, Wg21, bg21,
           Wp0, bp0, gamp, betp, Wp1, bp1):
    src = edge_index[0].reshape(NC, NS, CH, K)
    dst = edge_index[1]
    zeros = jnp.zeros((N, H), jnp.float32)

    layers = [
        _fold_bn(Wg00, bg00, gam0, bet0, Wg01, bg01),
        _fold_bn(Wg10, bg10, gam1, bet1, Wg11, bg11),
        _fold_bn(Wg20, bg20, gam2, bet2, Wg21, bg21),
    ]

    h = x
    for w0t, b0e, w1t, b1e in layers[:2]:
        parts = _sc_agg(h, src, dst, zeros)
        h = _mlp(parts, w0t, b0e, w1t, b1e)

    batch3 = batch.reshape(NB, 1, BN)
    scalep = gamp / jnp.sqrt(1.0 + EPS_BN)
    wp0t = (Wp0 * scalep[:, None]).T
    bp0e = (bp0 * scalep + betp)[None, :]
    wp1t = jnp.zeros((H, H), jnp.float32).at[:, :C].set(Wp1.T)
    bp1e = jnp.zeros((1, H), jnp.float32).at[0, :C].set(bp1)

    w0t, b0e, w1t, b1e = layers[2]
    parts = _sc_agg(h, src, dst, zeros)
    y = _mlp_pool(batch3, parts, w0t, b0e, w1t, b1e, wp0t, bp0e, wp1t, bp1e)
    return y[:, :C]


# speedup vs baseline: 4.1528x; 1.0200x over previous
import functools

import jax
import jax.numpy as jnp
from jax import lax
from jax.experimental import pallas as pl
from jax.experimental.pallas import tpu as pltpu
from jax.experimental.pallas import tpu_sc as plsc

N = 10000
E = 320000
H = 128
G = 64
C = 10
EPS_BN = 1e-5

NC = 2
NS = 16
NW = NC * NS
K = 128
EPW = E // NW
CH = EPW // K
TK = EPW - CH * K
NROW = N
RPT = 624
TAIL = N - NS * RPT

_mesh = plsc.VectorSubcoreMesh(core_axis_name="c", subcore_axis_name="s")


@functools.partial(
    pl.kernel,
    out_type=jax.ShapeDtypeStruct((NC, N, H), jnp.float32),
    mesh=_mesh,
    scratch_types=[
        pltpu.VMEM((K,), jnp.int32),
        pltpu.VMEM((K,), jnp.int32),
        pltpu.VMEM((K,), jnp.int32),
        pltpu.VMEM((K,), jnp.int32),
        pltpu.VMEM((K,), jnp.int32),
        pltpu.VMEM((K,), jnp.int32),
        pltpu.VMEM((TK,), jnp.int32),
        pltpu.VMEM((TK,), jnp.int32),
        pltpu.VMEM((K, H), jnp.float32),
        pltpu.VMEM((K, H), jnp.float32),
        pltpu.VMEM((K, H), jnp.float32),
        pltpu.VMEM_SHARED((NROW, H), jnp.float32),
        pltpu.SemaphoreType.DMA,
        pltpu.SemaphoreType.DMA,
        pltpu.SemaphoreType.DMA,
        pltpu.SemaphoreType.DMA,
        pltpu.SemaphoreType.DMA,
        pltpu.SemaphoreType.DMA,
        pltpu.SemaphoreType.DMA,
        pltpu.SemaphoreType.DMA,
        pltpu.SemaphoreType.DMA,
        pltpu.SemaphoreType.DMA,
        pltpu.SemaphoreType.DMA,
        pltpu.SemaphoreType.DMA,
    ],
)
def _sc_agg(h_hbm, src_hbm, dst_hbm, zeros_hbm, out_hbm,
            s0, s1, s2, d0, d1, d2, st, dt, buf0, buf1, buf2, agg_sh,
            g0, g1, g2, si0, si1, si2, di0, di1, di2, ss0, ss1, ss2):
    c = lax.axis_index("c")
    s = lax.axis_index("s")
    eb = (c * NS + s) * EPW
    r0 = s * RPT
    @pl.when(c == 0)
    def _init_h():
        pltpu.sync_copy(h_hbm.at[pl.ds(r0, RPT)], agg_sh.at[pl.ds(r0, RPT)])

    @pl.when(c == 1)
    def _init_z():
        pltpu.sync_copy(zeros_hbm.at[pl.ds(r0, RPT)], agg_sh.at[pl.ds(r0, RPT)])

    @pl.when((c == 0) & (s == 0))
    def _tail_h():
        pltpu.sync_copy(h_hbm.at[pl.ds(NS * RPT, TAIL)],
                        agg_sh.at[pl.ds(NS * RPT, TAIL)])

    @pl.when((c == 1) & (s == 0))
    def _tail_z():
        pltpu.sync_copy(zeros_hbm.at[pl.ds(NS * RPT, TAIL)],
                        agg_sh.at[pl.ds(NS * RPT, TAIL)])

    S = (s0, s1, s2)
    D = (d0, d1, d2)
    B = (buf0, buf1, buf2)
    GS = (g0, g1, g2)
    SI = (si0, si1, si2)
    DI = (di0, di1, di2)
    SS = (ss0, ss1, ss2)

    def issue_sidx(i, p):
        pltpu.async_copy(src_hbm.at[pl.ds(eb + i * K, K)], S[p], SI[p])

    def wait_sidx(i, p):
        pltpu.make_async_copy(src_hbm.at[pl.ds(eb + i * K, K)], S[p], SI[p]).wait()

    def issue_didx(i, p):
        pltpu.async_copy(dst_hbm.at[pl.ds(eb + i * K, K)], D[p], DI[p])

    def wait_didx(i, p):
        pltpu.make_async_copy(dst_hbm.at[pl.ds(eb + i * K, K)], D[p], DI[p]).wait()

    def issue_gather(p):
        pltpu.async_copy(h_hbm.at[S[p]], B[p], GS[p])

    def wait_gather(p):
        pltpu.make_async_copy(h_hbm.at[S[p]], B[p], GS[p]).wait()

    def wait_scat(p):
        pltpu.make_async_copy(B[p], agg_sh.at[D[p]], SS[p]).wait()

    issue_sidx(0, 0)
    issue_sidx(1, 1)
    issue_didx(0, 0)
    wait_sidx(0, 0)
    issue_gather(0)
    plsc.subcore_barrier()

    def step(i, p, q, w):
        @pl.when(i >= 2)
        def _drain():
            wait_scat(q)

        @pl.when(i + 1 < CH)
        def _refill():
            issue_didx(i + 1, q)
            wait_sidx(i + 1, q)
            issue_gather(q)

        @pl.when(i + 2 < CH)
        def _sidx_ahead():
            issue_sidx(i + 2, w)

        wait_didx(i, p)
        wait_gather(p)
        pltpu.async_copy(B[p], agg_sh.at[D[p]], SS[p], add=True)

    def body(j, carry):
        i = 3 * j
        step(i, 0, 1, 2)
        step(i + 1, 1, 2, 0)
        step(i + 2, 2, 0, 1)
        return carry

    lax.fori_loop(0, CH // 3, body, 0)
    wait_scat(1)
    wait_scat(2)
    pltpu.sync_copy(src_hbm.at[pl.ds(eb + CH * K, TK)], st)
    pltpu.sync_copy(dst_hbm.at[pl.ds(eb + CH * K, TK)], dt)
    pltpu.async_copy(h_hbm.at[st], buf0.at[pl.ds(0, TK)], g0)
    pltpu.make_async_copy(h_hbm.at[st], buf0.at[pl.ds(0, TK)], g0).wait()
    pltpu.sync_copy(buf0.at[pl.ds(0, TK)], agg_sh.at[dt], add=True)
    plsc.subcore_barrier()
    pltpu.sync_copy(agg_sh.at[pl.ds(r0, RPT)], out_hbm.at[c, pl.ds(r0, RPT)])

    @pl.when(s == 0)
    def _write_tail():
        pltpu.sync_copy(agg_sh.at[pl.ds(NS * RPT, TAIL)],
                        out_hbm.at[c, pl.ds(NS * RPT, TAIL)])


BN = 2000
NB = N // BN


def _mlp_body(p_ref, w0_ref, b0_ref, w1_ref, b1_ref, out_ref):
    u = p_ref[0] + p_ref[1]
    y = jnp.dot(u, w0_ref[...], preferred_element_type=jnp.float32) + b0_ref[...]
    y = jnp.where(y >= 0.0, y, 0.1 * y)
    out_ref[...] = jnp.dot(y, w1_ref[...], preferred_element_type=jnp.float32) + b1_ref[...]


def _mlp(parts, w0t, b0, w1t, b1):
    return pl.pallas_call(
        _mlp_body,
        out_shape=jax.ShapeDtypeStruct((N, H), jnp.float32),
        grid=(NB,),
        in_specs=[
            pl.BlockSpec((NC, BN, H), lambda i: (0, i, 0)),
            pl.BlockSpec((H, H), lambda i: (0, 0)),
            pl.BlockSpec((1, H), lambda i: (0, 0)),
            pl.BlockSpec((H, H), lambda i: (0, 0)),
            pl.BlockSpec((1, H), lambda i: (0, 0)),
        ],
        out_specs=pl.BlockSpec((BN, H), lambda i: (i, 0)),
    )(parts, w0t, b0, w1t, b1)


def _mlp_pool_body(b_ref, p_ref, w0_ref, b0_ref, w1_ref, b1_ref,
                   wp0_ref, bp0_ref, wp1_ref, bp1_ref, out_ref, acc_ref):
    i = pl.program_id(0)

    @pl.when(i == 0)
    def _init():
        acc_ref[...] = jnp.zeros_like(acc_ref)

    u = p_ref[0] + p_ref[1]
    y = jnp.dot(u, w0_ref[...], preferred_element_type=jnp.float32) + b0_ref[...]
    y = jnp.where(y >= 0.0, y, 0.1 * y)
    h3 = jnp.dot(y, w1_ref[...], preferred_element_type=jnp.float32) + b1_ref[...]

    seg = b_ref[0]
    onehot = (lax.broadcasted_iota(jnp.int32, (G, BN), 0) == seg).astype(jnp.float32)
    acc_ref[...] += jnp.dot(onehot, h3, preferred_element_type=jnp.float32)

    @pl.when(i == NB - 1)
    def _fin():
        z = jnp.dot(acc_ref[...], wp0_ref[...], preferred_element_type=jnp.float32) + bp0_ref[...]
        z = jnp.where(z >= 0.0, z, 0.1 * z)
        out_ref[...] = jnp.dot(z, wp1_ref[...], preferred_element_type=jnp.float32) + bp1_ref[...]


def _mlp_pool(batch3, parts, w0t, b0, w1t, b1, wp0t, bp0, wp1t, bp1):
    return pl.pallas_call(
        _mlp_pool_body,
        out_shape=jax.ShapeDtypeStruct((G, H), jnp.float32),
        grid=(NB,),
        in_specs=[
            pl.BlockSpec((1, 1, BN), lambda i: (i, 0, 0)),
            pl.BlockSpec((NC, BN, H), lambda i: (0, i, 0)),
            pl.BlockSpec((H, H), lambda i: (0, 0)),
            pl.BlockSpec((1, H), lambda i: (0, 0)),
            pl.BlockSpec((H, H), lambda i: (0, 0)),
            pl.BlockSpec((1, H), lambda i: (0, 0)),
            pl.BlockSpec((H, H), lambda i: (0, 0)),
            pl.BlockSpec((1, H), lambda i: (0, 0)),
            pl.BlockSpec((H, H), lambda i: (0, 0)),
            pl.BlockSpec((1, H), lambda i: (0, 0)),
        ],
        out_specs=pl.BlockSpec((G, H), lambda i: (0, 0)),
        scratch_shapes=[pltpu.VMEM((G, H), jnp.float32)],
    )(batch3, parts, w0t, b0, w1t, b1, wp0t, bp0, wp1t, bp1)


def _fold_bn(W0, b0, gam, bet, W1, b1):
    scale = gam / jnp.sqrt(1.0 + EPS_BN)
    w0t = (W0 * scale[:, None]).T
    b0e = (b0 * scale + bet)[None, :]
    return w0t, b0e, W1.T, b1[None, :]


def kernel(x, edge_index, batch,
           Wg00, bg00, gam0, bet0, Wg01, bg01,
           Wg10, bg10, gam1, bet1, Wg11, bg11,
           Wg20, bg20, gam2, bet2, Wg21, bg21,
           Wp0, bp0, gamp, betp, Wp1, bp1):
    src = edge_index[0]
    dst = edge_index[1]
    zeros = jnp.zeros((N, H), jnp.float32)

    layers = [
        _fold_bn(Wg00, bg00, gam0, bet0, Wg01, bg01),
        _fold_bn(Wg10, bg10, gam1, bet1, Wg11, bg11),
        _fold_bn(Wg20, bg20, gam2, bet2, Wg21, bg21),
    ]

    h = x
    for w0t, b0e, w1t, b1e in layers[:2]:
        parts = _sc_agg(h, src, dst, zeros)
        h = _mlp(parts, w0t, b0e, w1t, b1e)

    batch3 = batch.reshape(NB, 1, BN)
    scalep = gamp / jnp.sqrt(1.0 + EPS_BN)
    wp0t = (Wp0 * scalep[:, None]).T
    bp0e = (bp0 * scalep + betp)[None, :]
    wp1t = jnp.zeros((H, H), jnp.float32).at[:, :C].set(Wp1.T)
    bp1e = jnp.zeros((1, H), jnp.float32).at[0, :C].set(bp1)

    w0t, b0e, w1t, b1e = layers[2]
    parts = _sc_agg(h, src, dst, zeros)
    y = _mlp_pool(batch3, parts, w0t, b0e, w1t, b1e, wp0t, bp0e, wp1t, bp1e)
    return y[:, :C]


# speedup vs baseline: 4.2766x; 1.0298x over previous
import functools

import jax
import jax.numpy as jnp
from jax import lax
from jax.experimental import pallas as pl
from jax.experimental.pallas import tpu as pltpu
from jax.experimental.pallas import tpu_sc as plsc

N = 10000
E = 320000
H = 128
G = 64
C = 10
EPS_BN = 1e-5

NC = 2
NS = 16
NW = NC * NS
K = 96
EPW = E // NW
CH = EPW // K
TK = EPW - CH * K
NROW = N
RPT = 624
TAIL = N - NS * RPT

_mesh = plsc.VectorSubcoreMesh(core_axis_name="c", subcore_axis_name="s")


@functools.partial(
    pl.kernel,
    out_type=jax.ShapeDtypeStruct((NC, N, H), jnp.float32),
    mesh=_mesh,
    scratch_types=[
        [pltpu.VMEM((K,), jnp.int32)] * 4,
        [pltpu.VMEM((K,), jnp.int32)] * 4,
        pltpu.VMEM((TK,), jnp.int32),
        pltpu.VMEM((TK,), jnp.int32),
        [pltpu.VMEM((K, H), jnp.float32)] * 4,
        pltpu.VMEM_SHARED((NROW, H), jnp.float32),
        [pltpu.SemaphoreType.DMA] * 4,
        [pltpu.SemaphoreType.DMA] * 4,
        [pltpu.SemaphoreType.DMA] * 4,
        [pltpu.SemaphoreType.DMA] * 4,
    ],
)
def _sc_agg(h_hbm, src_hbm, dst_hbm, zeros_hbm, out_hbm,
            S, D, st, dt, B, agg_sh, GS, SI, DI, SS):
    c = lax.axis_index("c")
    s = lax.axis_index("s")
    eb = (c * NS + s) * EPW
    r0 = s * RPT
    @pl.when(c == 0)
    def _init_h():
        pltpu.sync_copy(h_hbm.at[pl.ds(r0, RPT)], agg_sh.at[pl.ds(r0, RPT)])

    @pl.when(c == 1)
    def _init_z():
        pltpu.sync_copy(zeros_hbm.at[pl.ds(r0, RPT)], agg_sh.at[pl.ds(r0, RPT)])

    @pl.when((c == 0) & (s == 0))
    def _tail_h():
        pltpu.sync_copy(h_hbm.at[pl.ds(NS * RPT, TAIL)],
                        agg_sh.at[pl.ds(NS * RPT, TAIL)])

    @pl.when((c == 1) & (s == 0))
    def _tail_z():
        pltpu.sync_copy(zeros_hbm.at[pl.ds(NS * RPT, TAIL)],
                        agg_sh.at[pl.ds(NS * RPT, TAIL)])

    def issue_sidx(i, p):
        pltpu.async_copy(src_hbm.at[pl.ds(eb + i * K, K)], S[p], SI[p])

    def wait_sidx(i, p):
        pltpu.make_async_copy(src_hbm.at[pl.ds(eb + i * K, K)], S[p], SI[p]).wait()

    def issue_didx(i, p):
        pltpu.async_copy(dst_hbm.at[pl.ds(eb + i * K, K)], D[p], DI[p])

    def wait_didx(i, p):
        pltpu.make_async_copy(dst_hbm.at[pl.ds(eb + i * K, K)], D[p], DI[p]).wait()

    def issue_gather(p):
        pltpu.async_copy(h_hbm.at[S[p]], B[p], GS[p])

    def wait_gather(p):
        pltpu.make_async_copy(h_hbm.at[S[p]], B[p], GS[p]).wait()

    def wait_scat(p):
        pltpu.make_async_copy(B[p], agg_sh.at[D[p]], SS[p]).wait()

    issue_sidx(0, 0)
    issue_sidx(1, 1)
    issue_sidx(2, 2)
    issue_didx(0, 0)
    issue_didx(1, 1)
    wait_sidx(0, 0)
    issue_gather(0)
    wait_sidx(1, 1)
    issue_gather(1)
    plsc.subcore_barrier()

    def step(i, p, q, w):
        @pl.when(i >= 2)
        def _drain():
            wait_scat(q)

        @pl.when(i + 2 < CH)
        def _refill():
            issue_didx(i + 2, q)
            wait_sidx(i + 2, q)
            issue_gather(q)

        @pl.when(i + 3 < CH)
        def _sidx_ahead():
            issue_sidx(i + 3, w)

        wait_didx(i, p)
        wait_gather(p)
        pltpu.async_copy(B[p], agg_sh.at[D[p]], SS[p], add=True)

    def body(j, carry):
        i = 4 * j
        step(i, 0, 2, 3)
        step(i + 1, 1, 3, 0)
        step(i + 2, 2, 0, 1)
        step(i + 3, 3, 1, 2)
        return carry

    lax.fori_loop(0, CH // 4, body, 0)
    wait_scat(2)
    wait_scat(3)
    pltpu.sync_copy(src_hbm.at[pl.ds(eb + CH * K, TK)], st)
    pltpu.sync_copy(dst_hbm.at[pl.ds(eb + CH * K, TK)], dt)
    pltpu.async_copy(h_hbm.at[st], B[0].at[pl.ds(0, TK)], GS[0])
    pltpu.make_async_copy(h_hbm.at[st], B[0].at[pl.ds(0, TK)], GS[0]).wait()
    pltpu.sync_copy(B[0].at[pl.ds(0, TK)], agg_sh.at[dt], add=True)
    plsc.subcore_barrier()
    pltpu.sync_copy(agg_sh.at[pl.ds(r0, RPT)], out_hbm.at[c, pl.ds(r0, RPT)])

    @pl.when(s == 0)
    def _write_tail():
        pltpu.sync_copy(agg_sh.at[pl.ds(NS * RPT, TAIL)],
                        out_hbm.at[c, pl.ds(NS * RPT, TAIL)])


BN = 2000
NB = N // BN


def _mlp_body(p_ref, w0_ref, b0_ref, w1_ref, b1_ref, out_ref):
    u = p_ref[0] + p_ref[1]
    y = jnp.dot(u, w0_ref[...], preferred_element_type=jnp.float32) + b0_ref[...]
    y = jnp.where(y >= 0.0, y, 0.1 * y)
    out_ref[...] = jnp.dot(y, w1_ref[...], preferred_element_type=jnp.float32) + b1_ref[...]


def _mlp(parts, w0t, b0, w1t, b1):
    return pl.pallas_call(
        _mlp_body,
        out_shape=jax.ShapeDtypeStruct((N, H), jnp.float32),
        grid=(NB,),
        in_specs=[
            pl.BlockSpec((NC, BN, H), lambda i: (0, i, 0)),
            pl.BlockSpec((H, H), lambda i: (0, 0)),
            pl.BlockSpec((1, H), lambda i: (0, 0)),
            pl.BlockSpec((H, H), lambda i: (0, 0)),
            pl.BlockSpec((1, H), lambda i: (0, 0)),
        ],
        out_specs=pl.BlockSpec((BN, H), lambda i: (i, 0)),
    )(parts, w0t, b0, w1t, b1)


def _mlp_pool_body(b_ref, p_ref, w0_ref, b0_ref, w1_ref, b1_ref,
                   wp0_ref, bp0_ref, wp1_ref, bp1_ref, out_ref, acc_ref):
    i = pl.program_id(0)

    @pl.when(i == 0)
    def _init():
        acc_ref[...] = jnp.zeros_like(acc_ref)

    u = p_ref[0] + p_ref[1]
    y = jnp.dot(u, w0_ref[...], preferred_element_type=jnp.float32) + b0_ref[...]
    y = jnp.where(y >= 0.0, y, 0.1 * y)
    h3 = jnp.dot(y, w1_ref[...], preferred_element_type=jnp.float32) + b1_ref[...]

    seg = b_ref[0]
    onehot = (lax.broadcasted_iota(jnp.int32, (G, BN), 0) == seg).astype(jnp.float32)
    acc_ref[...] += jnp.dot(onehot, h3, preferred_element_type=jnp.float32)

    @pl.when(i == NB - 1)
    def _fin():
        z = jnp.dot(acc_ref[...], wp0_ref[...], preferred_element_type=jnp.float32) + bp0_ref[...]
        z = jnp.where(z >= 0.0, z, 0.1 * z)
        out_ref[...] = jnp.dot(z, wp1_ref[...], preferred_element_type=jnp.float32) + bp1_ref[...]


def _mlp_pool(batch3, parts, w0t, b0, w1t, b1, wp0t, bp0, wp1t, bp1):
    return pl.pallas_call(
        _mlp_pool_body,
        out_shape=jax.ShapeDtypeStruct((G, H), jnp.float32),
        grid=(NB,),
        in_specs=[
            pl.BlockSpec((1, 1, BN), lambda i: (i, 0, 0)),
            pl.BlockSpec((NC, BN, H), lambda i: (0, i, 0)),
            pl.BlockSpec((H, H), lambda i: (0, 0)),
            pl.BlockSpec((1, H), lambda i: (0, 0)),
            pl.BlockSpec((H, H), lambda i: (0, 0)),
            pl.BlockSpec((1, H), lambda i: (0, 0)),
            pl.BlockSpec((H, H), lambda i: (0, 0)),
            pl.BlockSpec((1, H), lambda i: (0, 0)),
            pl.BlockSpec((H, H), lambda i: (0, 0)),
            pl.BlockSpec((1, H), lambda i: (0, 0)),
        ],
        out_specs=pl.BlockSpec((G, H), lambda i: (0, 0)),
        scratch_shapes=[pltpu.VMEM((G, H), jnp.float32)],
    )(batch3, parts, w0t, b0, w1t, b1, wp0t, bp0, wp1t, bp1)


def _fold_bn(W0, b0, gam, bet, W1, b1):
    scale = gam / jnp.sqrt(1.0 + EPS_BN)
    w0t = (W0 * scale[:, None]).T
    b0e = (b0 * scale + bet)[None, :]
    return w0t, b0e, W1.T, b1[None, :]


def kernel(x, edge_index, batch,
           Wg00, bg00, gam0, bet0, Wg01, bg01,
           Wg10, bg10, gam1, bet1, Wg11, bg11,
           Wg20, bg20, gam2, bet2, Wg21, bg21,
           Wp0, bp0, gamp, betp, Wp1, bp1):
    src = edge_index[0]
    dst = edge_index[1]
    zeros = jnp.zeros((N, H), jnp.float32)

    layers = [
        _fold_bn(Wg00, bg00, gam0, bet0, Wg01, bg01),
        _fold_bn(Wg10, bg10, gam1, bet1, Wg11, bg11),
        _fold_bn(Wg20, bg20, gam2, bet2, Wg21, bg21),
    ]

    h = x
    for w0t, b0e, w1t, b1e in layers[:2]:
        parts = _sc_agg(h, src, dst, zeros)
        h = _mlp(parts, w0t, b0e, w1t, b1e)

    batch3 = batch.reshape(NB, 1, BN)
    scalep = gamp / jnp.sqrt(1.0 + EPS_BN)
    wp0t = (Wp0 * scalep[:, None]).T
    bp0e = (bp0 * scalep + betp)[None, :]
    wp1t = jnp.zeros((H, H), jnp.float32).at[:, :C].set(Wp1.T)
    bp1e = jnp.zeros((1, H), jnp.float32).at[0, :C].set(bp1)

    w0t, b0e, w1t, b1e = layers[2]
    parts = _sc_agg(h, src, dst, zeros)
    y = _mlp_pool(batch3, parts, w0t, b0e, w1t, b1e, wp0t, bp0e, wp1t, bp1e)
    return y[:, :C]


# speedup vs baseline: 4.3430x; 1.0155x over previous
import functools

import jax
import jax.numpy as jnp
from jax import lax
from jax.experimental import pallas as pl
from jax.experimental.pallas import tpu as pltpu
from jax.experimental.pallas import tpu_sc as plsc

N = 10000
E = 320000
H = 128
G = 64
C = 10
EPS_BN = 1e-5

NC = 2
NS = 16
NW = NC * NS
K = 96
EPW = E // NW
CH = EPW // K
TK = EPW - CH * K
NROW = N
RPT = 624
TAIL = N - NS * RPT

_mesh = plsc.VectorSubcoreMesh(core_axis_name="c", subcore_axis_name="s")


@functools.partial(
    pl.kernel,
    out_type=jax.ShapeDtypeStruct((NC, N, H), jnp.float32),
    mesh=_mesh,
    scratch_types=[
        [pltpu.VMEM((K,), jnp.int32)] * 4,
        [pltpu.VMEM((K,), jnp.int32)] * 4,
        pltpu.VMEM((TK,), jnp.int32),
        pltpu.VMEM((TK,), jnp.int32),
        [pltpu.VMEM((K, H), jnp.float32)] * 4,
        pltpu.VMEM_SHARED((NROW, H), jnp.float32),
        [pltpu.SemaphoreType.DMA] * 4,
        [pltpu.SemaphoreType.DMA] * 4,
        [pltpu.SemaphoreType.DMA] * 4,
        [pltpu.SemaphoreType.DMA] * 4,
    ],
)
def _sc_agg(h_hbm, src_hbm, dst_hbm, zeros_hbm, out_hbm,
            S, D, st, dt, B, agg_sh, GS, SI, DI, SS):
    c = lax.axis_index("c")
    s = lax.axis_index("s")
    eb = (c * NS + s) * EPW
    r0 = s * RPT
    init_src = (h_hbm, zeros_hbm)

    @pl.when(c == 0)
    def _init_h():
        pltpu.async_copy(h_hbm.at[pl.ds(r0, RPT)], agg_sh.at[pl.ds(r0, RPT)], SS[0])

    @pl.when(c == 1)
    def _init_z():
        pltpu.async_copy(zeros_hbm.at[pl.ds(r0, RPT)], agg_sh.at[pl.ds(r0, RPT)], SS[0])

    @pl.when((c == 0) & (s == 0))
    def _tail_h():
        pltpu.async_copy(h_hbm.at[pl.ds(NS * RPT, TAIL)],
                         agg_sh.at[pl.ds(NS * RPT, TAIL)], SS[1])

    @pl.when((c == 1) & (s == 0))
    def _tail_z():
        pltpu.async_copy(zeros_hbm.at[pl.ds(NS * RPT, TAIL)],
                         agg_sh.at[pl.ds(NS * RPT, TAIL)], SS[1])

    def issue_sidx(i, p):
        pltpu.async_copy(src_hbm.at[pl.ds(eb + i * K, K)], S[p], SI[p])

    def wait_sidx(i, p):
        pltpu.make_async_copy(src_hbm.at[pl.ds(eb + i * K, K)], S[p], SI[p]).wait()

    def issue_didx(i, p):
        pltpu.async_copy(dst_hbm.at[pl.ds(eb + i * K, K)], D[p], DI[p])

    def wait_didx(i, p):
        pltpu.make_async_copy(dst_hbm.at[pl.ds(eb + i * K, K)], D[p], DI[p]).wait()

    def issue_gather(p):
        pltpu.async_copy(h_hbm.at[S[p]], B[p], GS[p])

    def wait_gather(p):
        pltpu.make_async_copy(h_hbm.at[S[p]], B[p], GS[p]).wait()

    def wait_scat(p):
        pltpu.make_async_copy(B[p], agg_sh.at[D[p]], SS[p]).wait()

    issue_sidx(0, 0)
    issue_sidx(1, 1)
    issue_sidx(2, 2)
    issue_didx(0, 0)
    issue_didx(1, 1)
    wait_sidx(0, 0)
    issue_gather(0)
    wait_sidx(1, 1)
    issue_gather(1)

    @pl.when(c == 0)
    def _winit_h():
        pltpu.make_async_copy(h_hbm.at[pl.ds(r0, RPT)],
                              agg_sh.at[pl.ds(r0, RPT)], SS[0]).wait()

    @pl.when(c == 1)
    def _winit_z():
        pltpu.make_async_copy(zeros_hbm.at[pl.ds(r0, RPT)],
                              agg_sh.at[pl.ds(r0, RPT)], SS[0]).wait()

    @pl.when((c == 0) & (s == 0))
    def _wtail_h():
        pltpu.make_async_copy(h_hbm.at[pl.ds(NS * RPT, TAIL)],
                              agg_sh.at[pl.ds(NS * RPT, TAIL)], SS[1]).wait()

    @pl.when((c == 1) & (s == 0))
    def _wtail_z():
        pltpu.make_async_copy(zeros_hbm.at[pl.ds(NS * RPT, TAIL)],
                              agg_sh.at[pl.ds(NS * RPT, TAIL)], SS[1]).wait()

    plsc.subcore_barrier()

    def step(i, p, q, w):
        @pl.when(i >= 2)
        def _drain():
            wait_scat(q)

        @pl.when(i + 2 < CH)
        def _refill():
            issue_didx(i + 2, q)
            wait_sidx(i + 2, q)
            issue_gather(q)

        @pl.when(i + 3 < CH)
        def _sidx_ahead():
            issue_sidx(i + 3, w)

        wait_didx(i, p)
        wait_gather(p)
        pltpu.async_copy(B[p], agg_sh.at[D[p]], SS[p], add=True)

    def body(j, carry):
        i = 4 * j
        step(i, 0, 2, 3)
        step(i + 1, 1, 3, 0)
        step(i + 2, 2, 0, 1)
        step(i + 3, 3, 1, 2)
        return carry

    lax.fori_loop(0, CH // 4, body, 0)
    wait_scat(2)
    wait_scat(3)
    pltpu.sync_copy(src_hbm.at[pl.ds(eb + CH * K, TK)], st)
    pltpu.sync_copy(dst_hbm.at[pl.ds(eb + CH * K, TK)], dt)
    pltpu.async_copy(h_hbm.at[st], B[0].at[pl.ds(0, TK)], GS[0])
    pltpu.make_async_copy(h_hbm.at[st], B[0].at[pl.ds(0, TK)], GS[0]).wait()
    pltpu.sync_copy(B[0].at[pl.ds(0, TK)], agg_sh.at[dt], add=True)
    plsc.subcore_barrier()
    pltpu.sync_copy(agg_sh.at[pl.ds(r0, RPT)], out_hbm.at[c, pl.ds(r0, RPT)])

    @pl.when(s == 0)
    def _write_tail():
        pltpu.sync_copy(agg_sh.at[pl.ds(NS * RPT, TAIL)],
                        out_hbm.at[c, pl.ds(NS * RPT, TAIL)])


BN = 2000
NB = N // BN


def _mlp_body(p_ref, w0_ref, b0_ref, w1_ref, b1_ref, out_ref):
    u = p_ref[0] + p_ref[1]
    y = jnp.dot(u, w0_ref[...], preferred_element_type=jnp.float32) + b0_ref[...]
    y = jnp.where(y >= 0.0, y, 0.1 * y)
    out_ref[...] = jnp.dot(y, w1_ref[...], preferred_element_type=jnp.float32) + b1_ref[...]


def _mlp(parts, w0t, b0, w1t, b1):
    return pl.pallas_call(
        _mlp_body,
        out_shape=jax.ShapeDtypeStruct((N, H), jnp.float32),
        grid=(NB,),
        in_specs=[
            pl.BlockSpec((NC, BN, H), lambda i: (0, i, 0)),
            pl.BlockSpec((H, H), lambda i: (0, 0)),
            pl.BlockSpec((1, H), lambda i: (0, 0)),
            pl.BlockSpec((H, H), lambda i: (0, 0)),
            pl.BlockSpec((1, H), lambda i: (0, 0)),
        ],
        out_specs=pl.BlockSpec((BN, H), lambda i: (i, 0)),
    )(parts, w0t, b0, w1t, b1)


def _mlp_pool_body(b_ref, p_ref, w0_ref, b0_ref, w1_ref, b1_ref,
                   wp0_ref, bp0_ref, wp1_ref, bp1_ref, out_ref, acc_ref):
    i = pl.program_id(0)

    @pl.when(i == 0)
    def _init():
        acc_ref[...] = jnp.zeros_like(acc_ref)

    u = p_ref[0] + p_ref[1]
    y = jnp.dot(u, w0_ref[...], preferred_element_type=jnp.float32) + b0_ref[...]
    y = jnp.where(y >= 0.0, y, 0.1 * y)
    h3 = jnp.dot(y, w1_ref[...], preferred_element_type=jnp.float32) + b1_ref[...]

    seg = b_ref[0]
    onehot = (lax.broadcasted_iota(jnp.int32, (G, BN), 0) == seg).astype(jnp.float32)
    acc_ref[...] += jnp.dot(onehot, h3, preferred_element_type=jnp.float32)

    @pl.when(i == NB - 1)
    def _fin():
        z = jnp.dot(acc_ref[...], wp0_ref[...], preferred_element_type=jnp.float32) + bp0_ref[...]
        z = jnp.where(z >= 0.0, z, 0.1 * z)
        out_ref[...] = jnp.dot(z, wp1_ref[...], preferred_element_type=jnp.float32) + bp1_ref[...]


def _mlp_pool(batch3, parts, w0t, b0, w1t, b1, wp0t, bp0, wp1t, bp1):
    return pl.pallas_call(
        _mlp_pool_body,
        out_shape=jax.ShapeDtypeStruct((G, H), jnp.float32),
        grid=(NB,),
        in_specs=[
            pl.BlockSpec((1, 1, BN), lambda i: (i, 0, 0)),
            pl.BlockSpec((NC, BN, H), lambda i: (0, i, 0)),
            pl.BlockSpec((H, H), lambda i: (0, 0)),
            pl.BlockSpec((1, H), lambda i: (0, 0)),
            pl.BlockSpec((H, H), lambda i: (0, 0)),
            pl.BlockSpec((1, H), lambda i: (0, 0)),
            pl.BlockSpec((H, H), lambda i: (0, 0)),
            pl.BlockSpec((1, H), lambda i: (0, 0)),
            pl.BlockSpec((H, H), lambda i: (0, 0)),
            pl.BlockSpec((1, H), lambda i: (0, 0)),
        ],
        out_specs=pl.BlockSpec((G, H), lambda i: (0, 0)),
        scratch_shapes=[pltpu.VMEM((G, H), jnp.float32)],
    )(batch3, parts, w0t, b0, w1t, b1, wp0t, bp0, wp1t, bp1)


def _fold_bn(W0, b0, gam, bet, W1, b1):
    scale = gam / jnp.sqrt(1.0 + EPS_BN)
    w0t = (W0 * scale[:, None]).T
    b0e = (b0 * scale + bet)[None, :]
    return w0t, b0e, W1.T, b1[None, :]


def kernel(x, edge_index, batch,
           Wg00, bg00, gam0, bet0, Wg01, bg01,
           Wg10, bg10, gam1, bet1, Wg11, bg11,
           Wg20, bg20, gam2, bet2, Wg21, bg21,
           Wp0, bp0, gamp, betp, Wp1, bp1):
    src = edge_index[0]
    dst = edge_index[1]
    zeros = jnp.zeros((N, H), jnp.float32)

    layers = [
        _fold_bn(Wg00, bg00, gam0, bet0, Wg01, bg01),
        _fold_bn(Wg10, bg10, gam1, bet1, Wg11, bg11),
        _fold_bn(Wg20, bg20, gam2, bet2, Wg21, bg21),
    ]

    h = x
    for w0t, b0e, w1t, b1e in layers[:2]:
        parts = _sc_agg(h, src, dst, zeros)
        h = _mlp(parts, w0t, b0e, w1t, b1e)

    batch3 = batch.reshape(NB, 1, BN)
    scalep = gamp / jnp.sqrt(1.0 + EPS_BN)
    wp0t = (Wp0 * scalep[:, None]).T
    bp0e = (bp0 * scalep + betp)[None, :]
    wp1t = jnp.zeros((H, H), jnp.float32).at[:, :C].set(Wp1.T)
    bp1e = jnp.zeros((1, H), jnp.float32).at[0, :C].set(bp1)

    w0t, b0e, w1t, b1e = layers[2]
    parts = _sc_agg(h, src, dst, zeros)
    y = _mlp_pool(batch3, parts, w0t, b0e, w1t, b1e, wp0t, bp0e, wp1t, bp1e)
    return y[:, :C]


# speedup vs baseline: 4.3474x; 1.0010x over previous
import functools

import jax
import jax.numpy as jnp
from jax import lax
from jax.experimental import pallas as pl
from jax.experimental.pallas import tpu as pltpu
from jax.experimental.pallas import tpu_sc as plsc

N = 10000
E = 320000
H = 128
G = 64
C = 10
EPS_BN = 1e-5

NC = 2
NS = 16
NW = NC * NS
K = 96
EPW = E // NW
CH = EPW // K
TK = EPW - CH * K
NROW = N
RPT = 624
TAIL = N - NS * RPT

_mesh = plsc.VectorSubcoreMesh(core_axis_name="c", subcore_axis_name="s")


@functools.partial(
    pl.kernel,
    out_type=jax.ShapeDtypeStruct((NC, N, H), jnp.float32),
    mesh=_mesh,
    scratch_types=[
        [pltpu.VMEM((K,), jnp.int32)] * 4,
        [pltpu.VMEM((K,), jnp.int32)] * 4,
        pltpu.VMEM((TK,), jnp.int32),
        pltpu.VMEM((TK,), jnp.int32),
        [pltpu.VMEM((K, H), jnp.float32)] * 4,
        pltpu.VMEM_SHARED((NROW, H), jnp.float32),
        [pltpu.SemaphoreType.DMA] * 4,
        [pltpu.SemaphoreType.DMA] * 4,
        [pltpu.SemaphoreType.DMA] * 4,
        [pltpu.SemaphoreType.DMA] * 4,
    ],
)
def _sc_agg(h_hbm, src_hbm, dst_hbm, zeros_hbm, out_hbm,
            S, D, st, dt, B, agg_sh, GS, SI, DI, SS):
    c = lax.axis_index("c")
    s = lax.axis_index("s")
    eb = (c * NS + s) * EPW
    r0 = s * RPT
    @pl.when(c == 0)
    def _init_h():
        pltpu.async_copy(h_hbm.at[pl.ds(r0, RPT)], agg_sh.at[pl.ds(r0, RPT)], SS[0])

    @pl.when(c == 1)
    def _init_z():
        pltpu.async_copy(zeros_hbm.at[pl.ds(r0, RPT)], agg_sh.at[pl.ds(r0, RPT)], SS[0])

    @pl.when((c == 0) & (s == 0))
    def _tail_h():
        pltpu.async_copy(h_hbm.at[pl.ds(NS * RPT, TAIL)],
                         agg_sh.at[pl.ds(NS * RPT, TAIL)], SS[1])

    @pl.when((c == 1) & (s == 0))
    def _tail_z():
        pltpu.async_copy(zeros_hbm.at[pl.ds(NS * RPT, TAIL)],
                         agg_sh.at[pl.ds(NS * RPT, TAIL)], SS[1])

    def issue_sidx(i, p):
        pltpu.async_copy(src_hbm.at[pl.ds(eb + i * K, K)], S[p], SI[p])

    def wait_sidx(i, p):
        pltpu.make_async_copy(src_hbm.at[pl.ds(eb + i * K, K)], S[p], SI[p]).wait()

    def issue_didx(i, p):
        pltpu.async_copy(dst_hbm.at[pl.ds(eb + i * K, K)], D[p], DI[p])

    def wait_didx(i, p):
        pltpu.make_async_copy(dst_hbm.at[pl.ds(eb + i * K, K)], D[p], DI[p]).wait()

    def issue_gather(p):
        pltpu.async_copy(h_hbm.at[S[p]], B[p], GS[p])

    def wait_gather(p):
        pltpu.make_async_copy(h_hbm.at[S[p]], B[p], GS[p]).wait()

    def wait_scat(p):
        pltpu.make_async_copy(B[p], agg_sh.at[D[p]], SS[p]).wait()

    issue_sidx(0, 0)
    issue_sidx(1, 1)
    issue_sidx(2, 2)
    issue_didx(0, 0)
    issue_didx(1, 1)
    wait_sidx(0, 0)
    issue_gather(0)
    wait_sidx(1, 1)
    issue_gather(1)

    @pl.when(c == 0)
    def _winit_h():
        pltpu.make_async_copy(h_hbm.at[pl.ds(r0, RPT)],
                              agg_sh.at[pl.ds(r0, RPT)], SS[0]).wait()

    @pl.when(c == 1)
    def _winit_z():
        pltpu.make_async_copy(zeros_hbm.at[pl.ds(r0, RPT)],
                              agg_sh.at[pl.ds(r0, RPT)], SS[0]).wait()

    @pl.when((c == 0) & (s == 0))
    def _wtail_h():
        pltpu.make_async_copy(h_hbm.at[pl.ds(NS * RPT, TAIL)],
                              agg_sh.at[pl.ds(NS * RPT, TAIL)], SS[1]).wait()

    @pl.when((c == 1) & (s == 0))
    def _wtail_z():
        pltpu.make_async_copy(zeros_hbm.at[pl.ds(NS * RPT, TAIL)],
                              agg_sh.at[pl.ds(NS * RPT, TAIL)], SS[1]).wait()

    plsc.subcore_barrier()

    def step(i, p, q, w):
        @pl.when(i >= 2)
        def _drain():
            wait_scat(q)

        @pl.when(i + 2 < CH)
        def _refill():
            issue_didx(i + 2, q)
            wait_sidx(i + 2, q)
            issue_gather(q)

        @pl.when(i + 3 < CH)
        def _sidx_ahead():
            issue_sidx(i + 3, w)

        wait_didx(i, p)
        wait_gather(p)
        pltpu.async_copy(B[p], agg_sh.at[D[p]], SS[p], add=True)

    def body(j, carry):
        i = 4 * j
        step(i, 0, 2, 3)
        step(i + 1, 1, 3, 0)
        step(i + 2, 2, 0, 1)
        step(i + 3, 3, 1, 2)
        return carry

    lax.fori_loop(0, CH // 4, body, 0)
    wait_scat(2)
    wait_scat(3)
    pltpu.sync_copy(src_hbm.at[pl.ds(eb + CH * K, TK)], st)
    pltpu.sync_copy(dst_hbm.at[pl.ds(eb + CH * K, TK)], dt)
    pltpu.async_copy(h_hbm.at[st], B[0].at[pl.ds(0, TK)], GS[0])
    pltpu.make_async_copy(h_hbm.at[st], B[0].at[pl.ds(0, TK)], GS[0]).wait()
    pltpu.sync_copy(B[0].at[pl.ds(0, TK)], agg_sh.at[dt], add=True)
    plsc.subcore_barrier()
    pltpu.sync_copy(agg_sh.at[pl.ds(r0, RPT)], out_hbm.at[c, pl.ds(r0, RPT)])

    @pl.when(s == 0)
    def _write_tail():
        pltpu.sync_copy(agg_sh.at[pl.ds(NS * RPT, TAIL)],
                        out_hbm.at[c, pl.ds(NS * RPT, TAIL)])


BN = 2000
NB = N // BN


def _mlp_body(p_ref, w0_ref, b0_ref, w1_ref, b1_ref, out_ref):
    u = p_ref[0] + p_ref[1]
    y = jnp.dot(u, w0_ref[...], preferred_element_type=jnp.float32) + b0_ref[...]
    y = jnp.where(y >= 0.0, y, 0.1 * y)
    out_ref[...] = jnp.dot(y, w1_ref[...], preferred_element_type=jnp.float32) + b1_ref[...]


def _mlp(parts, w0t, b0, w1t, b1):
    return pl.pallas_call(
        _mlp_body,
        out_shape=jax.ShapeDtypeStruct((N, H), jnp.float32),
        grid=(NB,),
        in_specs=[
            pl.BlockSpec((NC, BN, H), lambda i: (0, i, 0)),
            pl.BlockSpec((H, H), lambda i: (0, 0)),
            pl.BlockSpec((1, H), lambda i: (0, 0)),
            pl.BlockSpec((H, H), lambda i: (0, 0)),
            pl.BlockSpec((1, H), lambda i: (0, 0)),
        ],
        out_specs=pl.BlockSpec((BN, H), lambda i: (i, 0)),
    )(parts, w0t, b0, w1t, b1)


def _mlp_pool_body(b_ref, p_ref, w0_ref, b0_ref, w1_ref, b1_ref,
                   wp0_ref, bp0_ref, wp1_ref, bp1_ref, out_ref, acc_ref):
    i = pl.program_id(0)

    @pl.when(i == 0)
    def _init():
        acc_ref[...] = jnp.zeros_like(acc_ref)

    u = p_ref[0] + p_ref[1]
    y = jnp.dot(u, w0_ref[...], preferred_element_type=jnp.float32) + b0_ref[...]
    y = jnp.where(y >= 0.0, y, 0.1 * y)
    h3 = jnp.dot(y, w1_ref[...], preferred_element_type=jnp.float32) + b1_ref[...]

    seg = b_ref[0]
    onehot = (lax.broadcasted_iota(jnp.int32, (G, BN), 0) == seg).astype(jnp.float32)
    acc_ref[...] += jnp.dot(onehot, h3, preferred_element_type=jnp.float32)

    @pl.when(i == NB - 1)
    def _fin():
        z = jnp.dot(acc_ref[...], wp0_ref[...], preferred_element_type=jnp.float32) + bp0_ref[...]
        z = jnp.where(z >= 0.0, z, 0.1 * z)
        out_ref[...] = jnp.dot(z, wp1_ref[...], preferred_element_type=jnp.float32) + bp1_ref[...]


def _mlp_pool(batch3, parts, w0t, b0, w1t, b1, wp0t, bp0, wp1t, bp1):
    return pl.pallas_call(
        _mlp_pool_body,
        out_shape=jax.ShapeDtypeStruct((G, H), jnp.float32),
        grid=(NB,),
        in_specs=[
            pl.BlockSpec((1, 1, BN), lambda i: (i, 0, 0)),
            pl.BlockSpec((NC, BN, H), lambda i: (0, i, 0)),
            pl.BlockSpec((H, H), lambda i: (0, 0)),
            pl.BlockSpec((1, H), lambda i: (0, 0)),
            pl.BlockSpec((H, H), lambda i: (0, 0)),
            pl.BlockSpec((1, H), lambda i: (0, 0)),
            pl.BlockSpec((H, H), lambda i: (0, 0)),
            pl.BlockSpec((1, H), lambda i: (0, 0)),
            pl.BlockSpec((H, H), lambda i: (0, 0)),
            pl.BlockSpec((1, H), lambda i: (0, 0)),
        ],
        out_specs=pl.BlockSpec((G, H), lambda i: (0, 0)),
        scratch_shapes=[pltpu.VMEM((G, H), jnp.float32)],
    )(batch3, parts, w0t, b0, w1t, b1, wp0t, bp0, wp1t, bp1)


def _fold_bn(W0, b0, gam, bet, W1, b1):
    scale = gam / jnp.sqrt(1.0 + EPS_BN)
    w0t = (W0 * scale[:, None]).T
    b0e = (b0 * scale + bet)[None, :]
    return w0t, b0e, W1.T, b1[None, :]


def kernel(x, edge_index, batch,
           Wg00, bg00, gam0, bet0, Wg01, bg01,
           Wg10, bg10, gam1, bet1, Wg11, bg11,
           Wg20, bg20, gam2, bet2, Wg21, bg21,
           Wp0, bp0, gamp, betp, Wp1, bp1):
    src = edge_index[0]
    dst = edge_index[1]
    zeros = jnp.zeros((N, H), jnp.float32)

    layers = [
        _fold_bn(Wg00, bg00, gam0, bet0, Wg01, bg01),
        _fold_bn(Wg10, bg10, gam1, bet1, Wg11, bg11),
        _fold_bn(Wg20, bg20, gam2, bet2, Wg21, bg21),
    ]

    h = x
    for w0t, b0e, w1t, b1e in layers[:2]:
        parts = _sc_agg(h, src, dst, zeros)
        h = _mlp(parts, w0t, b0e, w1t, b1e)

    batch3 = batch.reshape(NB, 1, BN)
    scalep = gamp / jnp.sqrt(1.0 + EPS_BN)
    wp0t = (Wp0 * scalep[:, None]).T
    bp0e = (bp0 * scalep + betp)[None, :]
    wp1t = jnp.zeros((H, H), jnp.float32).at[:, :C].set(Wp1.T)
    bp1e = jnp.zeros((1, H), jnp.float32).at[0, :C].set(bp1)

    w0t, b0e, w1t, b1e = layers[2]
    parts = _sc_agg(h, src, dst, zeros)
    y = _mlp_pool(batch3, parts, w0t, b0e, w1t, b1e, wp0t, bp0e, wp1t, bp1e)
    return y[:, :C]
